# baseline (speedup 1.0000x reference)
import functools

import jax
import jax.numpy as jnp
from jax import lax
from jax.experimental import pallas as pl
from jax.experimental.pallas import tpu as pltpu
from jax.experimental.pallas import tpu_sc as plsc

_NC, _NS, _LANES = 2, 16, 16
_NW = _NC * _NS
_CHUNK = 16
_RROWS = _CHUNK + 8
_NBUF = 3
_UNROLL = 8


def _make_sc_kernel(n_tokens, d, s_row, num_seq, seq_start):
    tpw = n_tokens // _NW
    nch = tpw // _CHUNK
    spans_per_row = s_row // tpw
    sent = num_seq + seq_start
    mesh = plsc.VectorSubcoreMesh(core_axis_name="c", subcore_axis_name="s")

    @functools.partial(
        pl.kernel,
        out_type=(jax.ShapeDtypeStruct((n_tokens, d), jnp.float32),
                  jax.ShapeDtypeStruct((_NW * _LANES,), jnp.int32)),
        mesh=mesh,
        compiler_params=pltpu.CompilerParams(needs_layout_passes=False),
        scratch_types=(
            [pltpu.VMEM((tpw,), jnp.int32),
             pltpu.VMEM((_LANES,), jnp.int32),
             pltpu.VMEM((tpw,), jnp.int32),
             pltpu.VMEM((tpw,), jnp.int32),
             pltpu.VMEM((seq_start, d), jnp.float32),
             pltpu.VMEM((_NW * _LANES,), jnp.int32)]
            + [pltpu.VMEM((_CHUNK, d), jnp.float32) for _ in range(_NBUF)]
            + [pltpu.VMEM((_RROWS, d), jnp.float32) for _ in range(_NBUF)]
            + [pltpu.SemaphoreType.DMA for _ in range(3 * _NBUF + 2)]
        ),
    )
    def sc_kernel(x_hbm, ids_hbm, table_hbm, ctrl_hbm, st_hbm,
                  out_hbm, exch_hbm,
                  ids_v, st_v, last_v, idx_v, ctrl_v, exch_v,
                  xb0, xb1, xb2, rb0, rb1, rb2, *sems):
        xbuf = [xb0, xb1, xb2]
        rbuf = [rb0, rb1, rb2]
        sx = sems[0:_NBUF]
        sg = sems[_NBUF:2 * _NBUF]
        so = sems[2 * _NBUF:3 * _NBUF]
        s_ctrl, s_fb = sems[3 * _NBUF], sems[3 * _NBUF + 1]

        iota = lax.broadcasted_iota(jnp.int32, (_LANES,), 0)
        cid = lax.axis_index("c")
        sid = lax.axis_index("s")
        wid = cid * _NS + sid
        base_tok = pl.multiple_of(wid * tpw, tpw)
        row = wid // spans_per_row
        pos0 = (wid % spans_per_row) * tpw

        def issue_x(j):
            pltpu.async_copy(
                x_hbm.at[pl.ds(base_tok + j * _CHUNK, _CHUNK)],
                xbuf[j % _NBUF], sx[j % _NBUF])

        issue_x(0)
        issue_x(1)
        pltpu.async_copy(ctrl_hbm, ctrl_v, s_ctrl)
        pltpu.sync_copy(ids_hbm.at[pl.ds(base_tok, tpw)], ids_v)
        pltpu.sync_copy(st_hbm, st_v)
        st = st_v[...]

        def scan_g(g, carry):
            v_ids = ids_v[pl.ds(g * _CHUNK, _LANES)]
            pos = pos0 + g * _CHUNK + iota
            marked = jnp.where((v_ids == st) & (pos >= seq_start), pos,
                               jnp.int32(-1))
            m = jnp.maximum(plsc.cummax(marked), carry)
            last_v[pl.ds(g * _CHUNK, _LANES)] = m
            return jnp.max(m)

        carry = lax.fori_loop(0, nch, scan_g, jnp.int32(-1))

        ids_v[pl.ds(0, _LANES)] = jnp.full((_LANES,), carry, jnp.int32)
        pltpu.sync_copy(ids_v.at[pl.ds(0, _LANES)],
                        exch_hbm.at[pl.ds(pl.multiple_of(wid * _LANES,
                                                         _LANES), _LANES)])
        plsc.subcore_barrier()
        pltpu.sync_copy(exch_hbm, exch_v)

        def fold(p, acc):
            return jnp.maximum(acc, jnp.max(exch_v[pl.ds(p * _LANES,
                                                         _LANES)]))

        prefix = lax.fori_loop(row * spans_per_row, wid, fold, jnp.int32(-1))

        @pl.loop(0, nch)
        def _fix(g):
            m = jnp.maximum(last_v[pl.ds(g * _CHUNK, _LANES)], prefix)
            pos = pos0 + g * _CHUNK + iota
            rel = pos - m
            valid = (m >= 0) & (rel < num_seq)
            idx_v[pl.ds(g * _CHUNK, _LANES)] = jnp.where(
                valid, rel,
                jnp.where(pos < seq_start, num_seq + pos, sent))

        flags = {}

        def issue_table(j):
            bi = j % _NBUF
            v = idx_v[pl.ds(j * _CHUNK, _LANES)]
            w = v - iota
            wmin, wmax = jnp.min(w), jnp.max(w)
            vmin, vmax = jnp.min(v), jnp.max(v)
            contig = wmax == wmin
            const = (vmax == vmin) & (vmin == jnp.int32(sent))
            base0 = jnp.minimum((wmin // 8) * 8, jnp.int32(num_seq - _RROWS))
            base0 = pl.multiple_of(base0, 8)

            @pl.when(contig)
            def _():
                pltpu.async_copy(table_hbm.at[pl.ds(base0, _RROWS)],
                                 rbuf[bi], sg[bi])

            @pl.when(jnp.logical_not(contig | const))
            def _():
                vcl = jnp.minimum(v, jnp.int32(num_seq - 1))
                pltpu.async_copy(table_hbm.at[vcl],
                                 rbuf[bi].at[pl.ds(0, _CHUNK)], s_fb).wait()

            off = jnp.where(contig, wmin - base0, jnp.int32(0))
            flags[j] = (contig, const, off)

        pltpu.make_async_copy(ctrl_hbm, ctrl_v, s_ctrl).wait()
        issue_table(0)
        issue_table(1)
        for j in range(nch):
            bi = j % _NBUF
            ri = j % _NBUF
            if j + 2 < nch:
                if j >= 1:
                    pltpu.make_async_copy(
                        xbuf[(j + 2) % _NBUF], out_hbm.at[pl.ds(0, _CHUNK)],
                        so[(j + 2) % _NBUF]).wait()
                issue_x(j + 2)
                issue_table(j + 2)
            contig, const, off = flags[j]
            pltpu.make_async_copy(
                x_hbm.at[pl.ds(0, _CHUNK)], xbuf[bi], sx[bi]).wait()

            @pl.when(contig)
            def _():
                pltpu.make_async_copy(
                    table_hbm.at[pl.ds(0, _RROWS)], rbuf[ri], sg[ri]).wait()

            xv, rv = xbuf[bi], rbuf[ri]

            @pl.when(jnp.logical_not(contig | const))
            def _():
                v = idx_v[pl.ds(j * _CHUNK, _LANES)]

                @pl.loop(0, _LANES)
                def _tok(k):
                    t = jnp.max(jnp.where(iota == k, v, jnp.int32(-1)))

                    @pl.when(t >= jnp.int32(sent))
                    def _():
                        @pl.loop(0, d, step=_LANES)
                        def _z(c):
                            rv[k, pl.ds(c, _LANES)] = jnp.zeros(
                                (_LANES,), jnp.float32)

                    @pl.when((t >= jnp.int32(num_seq)) &
                             (t < jnp.int32(sent)))
                    def _():
                        @pl.loop(0, d, step=_LANES)
                        def _c(c):
                            rv[k, pl.ds(c, _LANES)] = \
                                ctrl_v[t - num_seq, pl.ds(c, _LANES)]

            @pl.when(jnp.logical_not(const))
            def _():
                @pl.loop(0, _CHUNK)
                def _row(i):
                    @plsc.parallel_loop(0, d, step=_LANES, unroll=_UNROLL)
                    def _col(c):
                        sl = pl.ds(c, _LANES)
                        plsc.addupdate(xv.at[i, sl], rv[off + i, sl])

            pltpu.async_copy(xv, out_hbm.at[pl.ds(base_tok + j * _CHUNK,
                                                  _CHUNK)], so[bi])
        for jj in range(max(0, nch - 3), nch):
            pltpu.make_async_copy(
                xbuf[jj % _NBUF], out_hbm.at[pl.ds(0, _CHUNK)],
                so[jj % _NBUF]).wait()

    return sc_kernel


def kernel(x, input_ids, control_table, sequence_table, start_token):
    b, s, d = x.shape
    seq_start = control_table.shape[0]
    num_seq = sequence_table.shape[0]
    n = b * s
    ids = input_ids.astype(jnp.int32).reshape(n)
    st = jnp.full((_LANES,), start_token, jnp.int32)
    xf = x.reshape(n, d)
    out, _ = _make_sc_kernel(n, d, s, num_seq, seq_start)(
        xf, ids, sequence_table.astype(jnp.float32),
        control_table.astype(jnp.float32), st)
    return out.reshape(b, s, d)


# speedup vs baseline: 1.0249x; 1.0249x over previous
import functools

import jax
import jax.numpy as jnp
from jax import lax
from jax.experimental import pallas as pl
from jax.experimental.pallas import tpu as pltpu
from jax.experimental.pallas import tpu_sc as plsc

_NC, _NS, _LANES = 2, 16, 16
_NW = _NC * _NS
_CHUNK = 16
_RROWS = _CHUNK + 8
_NBUF = 3
_UNROLL = 8


def _make_sc_kernel(n_tokens, d, s_row, num_seq, seq_start):
    tpw = n_tokens // _NW
    nch = tpw // _CHUNK
    spans_per_row = s_row // tpw
    sent = num_seq + seq_start
    mesh = plsc.VectorSubcoreMesh(core_axis_name="c", subcore_axis_name="s")

    @functools.partial(
        pl.kernel,
        out_type=(jax.ShapeDtypeStruct((n_tokens, d), jnp.float32),
                  jax.ShapeDtypeStruct((_NW * _LANES,), jnp.int32)),
        mesh=mesh,
        compiler_params=pltpu.CompilerParams(needs_layout_passes=False),
        scratch_types=(
            [pltpu.VMEM((tpw,), jnp.int32),
             pltpu.VMEM((_LANES,), jnp.int32),
             pltpu.VMEM((tpw,), jnp.int32),
             pltpu.VMEM((tpw,), jnp.int32),
             pltpu.VMEM((seq_start, d), jnp.float32),
             pltpu.VMEM((_NW * _LANES,), jnp.int32)]
            + [pltpu.VMEM((_CHUNK, d), jnp.float32) for _ in range(_NBUF)]
            + [pltpu.VMEM((_RROWS, d), jnp.float32) for _ in range(2)]
            + [pltpu.SemaphoreType.DMA for _ in range(_NBUF + 2 + _NBUF + 2)]
        ),
    )
    def sc_kernel(x_hbm, ids_hbm, table_hbm, ctrl_hbm, st_hbm,
                  out_hbm, exch_hbm,
                  ids_v, st_v, last_v, idx_v, ctrl_v, exch_v,
                  xb0, xb1, xb2, rb0, rb1, *sems):
        xbuf = [xb0, xb1, xb2]
        rbuf = [rb0, rb1]
        sx = sems[0:_NBUF]
        sg = sems[_NBUF:_NBUF + 2]
        so = sems[_NBUF + 2:2 * _NBUF + 2]
        s_ctrl, s_fb = sems[2 * _NBUF + 2], sems[2 * _NBUF + 3]

        iota = lax.broadcasted_iota(jnp.int32, (_LANES,), 0)
        cid = lax.axis_index("c")
        sid = lax.axis_index("s")
        wid = cid * _NS + sid
        base_tok = pl.multiple_of(wid * tpw, tpw)
        row = wid // spans_per_row
        pos0 = (wid % spans_per_row) * tpw

        def issue_x(j):
            pltpu.async_copy(
                x_hbm.at[pl.ds(base_tok + j * _CHUNK, _CHUNK)],
                xbuf[j % _NBUF], sx[j % _NBUF])

        issue_x(0)
        issue_x(1)
        pltpu.async_copy(ctrl_hbm, ctrl_v, s_ctrl)
        pltpu.sync_copy(ids_hbm.at[pl.ds(base_tok, tpw)], ids_v)
        pltpu.sync_copy(st_hbm, st_v)
        st = st_v[...]

        def scan_g(g, carry):
            v_ids = ids_v[pl.ds(g * _CHUNK, _LANES)]
            pos = pos0 + g * _CHUNK + iota
            marked = jnp.where((v_ids == st) & (pos >= seq_start), pos,
                               jnp.int32(-1))
            m = jnp.maximum(plsc.cummax(marked), carry)
            last_v[pl.ds(g * _CHUNK, _LANES)] = m
            return jnp.max(m)

        carry = lax.fori_loop(0, nch, scan_g, jnp.int32(-1))

        ids_v[pl.ds(0, _LANES)] = jnp.full((_LANES,), carry, jnp.int32)
        pltpu.sync_copy(ids_v.at[pl.ds(0, _LANES)],
                        exch_hbm.at[pl.ds(pl.multiple_of(wid * _LANES,
                                                         _LANES), _LANES)])
        plsc.subcore_barrier()
        pltpu.sync_copy(exch_hbm, exch_v)

        def fold(p, acc):
            return jnp.maximum(acc, jnp.max(exch_v[pl.ds(p * _LANES,
                                                         _LANES)]))

        prefix = lax.fori_loop(row * spans_per_row, wid, fold, jnp.int32(-1))

        @pl.loop(0, nch)
        def _fix(g):
            m = jnp.maximum(last_v[pl.ds(g * _CHUNK, _LANES)], prefix)
            pos = pos0 + g * _CHUNK + iota
            rel = pos - m
            valid = (m >= 0) & (rel < num_seq)
            idx_v[pl.ds(g * _CHUNK, _LANES)] = jnp.where(
                valid, rel,
                jnp.where(pos < seq_start, num_seq + pos, sent))

        flags = {}

        def issue_table(j):
            bi = j % 2
            v = idx_v[pl.ds(j * _CHUNK, _LANES)]
            w = v - iota
            wmin, wmax = jnp.min(w), jnp.max(w)
            vmin, vmax = jnp.min(v), jnp.max(v)
            contig = wmax == wmin
            const = (vmax == vmin) & (vmin == jnp.int32(sent))
            base0 = jnp.minimum((wmin // 8) * 8, jnp.int32(num_seq - _RROWS))
            base0 = pl.multiple_of(base0, 8)

            @pl.when(contig)
            def _():
                pltpu.async_copy(table_hbm.at[pl.ds(base0, _RROWS)],
                                 rbuf[bi], sg[bi])

            @pl.when(jnp.logical_not(contig | const))
            def _():
                vcl = jnp.minimum(v, jnp.int32(num_seq - 1))
                pltpu.async_copy(table_hbm.at[vcl],
                                 rbuf[bi].at[pl.ds(0, _CHUNK)], s_fb).wait()

            off = jnp.where(contig, wmin - base0, jnp.int32(0))
            flags[j] = (contig, const, off)

        pltpu.make_async_copy(ctrl_hbm, ctrl_v, s_ctrl).wait()
        issue_table(0)
        for j in range(nch):
            bi = j % _NBUF
            ri = j % 2
            if j + 1 < nch:
                issue_table(j + 1)
            if j + 2 < nch:
                if j >= 1:
                    pltpu.make_async_copy(
                        xbuf[(j + 2) % _NBUF], out_hbm.at[pl.ds(0, _CHUNK)],
                        so[(j + 2) % _NBUF]).wait()
                issue_x(j + 2)
            contig, const, off = flags[j]
            pltpu.make_async_copy(
                x_hbm.at[pl.ds(0, _CHUNK)], xbuf[bi], sx[bi]).wait()

            @pl.when(contig)
            def _():
                pltpu.make_async_copy(
                    table_hbm.at[pl.ds(0, _RROWS)], rbuf[ri], sg[ri]).wait()

            xv, rv = xbuf[bi], rbuf[ri]

            @pl.when(jnp.logical_not(contig | const))
            def _():
                v = idx_v[pl.ds(j * _CHUNK, _LANES)]

                @pl.loop(0, _LANES)
                def _tok(k):
                    t = jnp.max(jnp.where(iota == k, v, jnp.int32(-1)))

                    @pl.when(t >= jnp.int32(sent))
                    def _():
                        @pl.loop(0, d, step=_LANES)
                        def _z(c):
                            rv[k, pl.ds(c, _LANES)] = jnp.zeros(
                                (_LANES,), jnp.float32)

                    @pl.when((t >= jnp.int32(num_seq)) &
                             (t < jnp.int32(sent)))
                    def _():
                        @pl.loop(0, d, step=_LANES)
                        def _c(c):
                            rv[k, pl.ds(c, _LANES)] = \
                                ctrl_v[t - num_seq, pl.ds(c, _LANES)]

            @pl.when(jnp.logical_not(const))
            def _():
                @pl.loop(0, _CHUNK)
                def _row(i):
                    @plsc.parallel_loop(0, d, step=_LANES, unroll=_UNROLL)
                    def _col(c):
                        sl = pl.ds(c, _LANES)
                        plsc.addupdate(xv.at[i, sl], rv[off + i, sl])

            pltpu.async_copy(xv, out_hbm.at[pl.ds(base_tok + j * _CHUNK,
                                                  _CHUNK)], so[bi])
        for jj in range(max(0, nch - 3), nch):
            pltpu.make_async_copy(
                xbuf[jj % _NBUF], out_hbm.at[pl.ds(0, _CHUNK)],
                so[jj % _NBUF]).wait()

    return sc_kernel


def kernel(x, input_ids, control_table, sequence_table, start_token):
    b, s, d = x.shape
    seq_start = control_table.shape[0]
    num_seq = sequence_table.shape[0]
    n = b * s
    ids = input_ids.astype(jnp.int32).reshape(n)
    st = jnp.full((_LANES,), start_token, jnp.int32)
    xf = x.reshape(n, d)
    out, _ = _make_sc_kernel(n, d, s, num_seq, seq_start)(
        xf, ids, sequence_table.astype(jnp.float32),
        control_table.astype(jnp.float32), st)
    return out.reshape(b, s, d)
